# baseline (speedup 1.0000x reference)
import functools

import jax
import jax.numpy as jnp
from jax import lax
from jax.experimental import pallas as pl
from jax.experimental.pallas import tpu as pltpu
from jax.experimental.pallas import tpu_sc as plsc

S, B, D = 2048, 4, 2048
NC, NS = 2, 16
NW = NC * NS
ROWS_W = S // NW
CHUNK = 4
NCHUNK = ROWS_W // CHUNK
NBUF = 3


@functools.partial(
    pl.kernel,
    mesh=plsc.VectorSubcoreMesh(core_axis_name="c", subcore_axis_name="s"),
    out_type=jax.ShapeDtypeStruct((S + 1, B, D), jnp.float32),
    scratch_types=[
        pltpu.VMEM((NBUF, CHUNK, B, D), jnp.float32),
        pltpu.VMEM((B,), jnp.int32),
        pltpu.VMEM((B, D), jnp.float32),
        pltpu.SemaphoreType.DMA((NBUF,)),
        pltpu.SemaphoreType.DMA((NBUF,)),
        pltpu.SemaphoreType.DMA,
    ],
)
def _sc_concat_embed(x_hbm, t_hbm, table_hbm, out_hbm,
                     buf, idx_v, rows_v, in_sems, out_sems, gsem):
    wid = lax.axis_index("s") * NC + lax.axis_index("c")
    base = wid * ROWS_W

    @pl.when(wid == 0)
    def _embed():
        pltpu.sync_copy(t_hbm, idx_v)
        pltpu.async_copy(table_hbm.at[idx_v], rows_v, gsem).wait()
        pltpu.sync_copy(rows_v, out_hbm.at[S])

    def in_cp(k):
        return pltpu.make_async_copy(
            x_hbm.at[pl.ds(base + k * CHUNK, CHUNK)], buf.at[k % NBUF],
            in_sems.at[k % NBUF])

    def out_cp(k):
        return pltpu.make_async_copy(
            buf.at[k % NBUF], out_hbm.at[pl.ds(base + k * CHUNK, CHUNK)],
            out_sems.at[k % NBUF])

    PF = NBUF - 1
    for k in range(min(PF, NCHUNK)):
        in_cp(k).start()
    out_waited = set()
    for k in range(NCHUNK):
        nk = k + PF
        if nk < NCHUNK:
            j = nk - NBUF
            if j >= 0:
                out_cp(j).wait()
                out_waited.add(j)
            in_cp(nk).start()
        in_cp(k).wait()
        out_cp(k).start()
    for k in range(NCHUNK):
        if k not in out_waited:
            out_cp(k).wait()


def kernel(x, t, table):
    return _sc_concat_embed(x, t, table)


# speedup vs baseline: 1.0996x; 1.0996x over previous
import functools

import jax
import jax.numpy as jnp
from jax import lax
from jax.experimental import pallas as pl
from jax.experimental.pallas import tpu as pltpu
from jax.experimental.pallas import tpu_sc as plsc

S, B, D = 2048, 4, 2048
BS = 128
N = S // BS


@functools.partial(
    pl.kernel,
    mesh=plsc.VectorSubcoreMesh(core_axis_name="c", subcore_axis_name="s"),
    out_type=jax.ShapeDtypeStruct((B, D), jnp.float32),
    scratch_types=[
        pltpu.VMEM((B,), jnp.int32),
        pltpu.VMEM((B, D), jnp.float32),
        pltpu.SemaphoreType.DMA,
    ],
)
def _sc_embed(t_hbm, table_hbm, emb_hbm, idx_v, rows_v, gsem):
    wid = lax.axis_index("s") * 2 + lax.axis_index("c")

    @pl.when(wid == 0)
    def _gather():
        pltpu.sync_copy(t_hbm, idx_v)
        pltpu.async_copy(table_hbm.at[idx_v], rows_v, gsem).wait()
        pltpu.sync_copy(rows_v, emb_hbm)


def _tc_copy_body(x_ref, out_ref):
    i = pl.program_id(0)

    @pl.when(i < N)
    def _copy():
        out_ref[...] = x_ref[...]


def _patch_body(emb_ref, big_ref, out_ref, sem):
    pltpu.make_async_copy(emb_ref, out_ref.at[S], sem).start()
    pltpu.make_async_copy(emb_ref, out_ref.at[S], sem).wait()


def kernel(x, t, table):
    t_emb = _sc_embed(t, table)
    big = pl.pallas_call(
        _tc_copy_body,
        grid=(N + 1,),
        out_shape=jax.ShapeDtypeStruct((S + 1, B, D), x.dtype),
        in_specs=[
            pl.BlockSpec((BS, B, D), lambda i: (jnp.minimum(i, N - 1), 0, 0)),
        ],
        out_specs=pl.BlockSpec((BS, B, D), lambda i: (i, 0, 0)),
    )(x)
    return pl.pallas_call(
        _patch_body,
        out_shape=jax.ShapeDtypeStruct((S + 1, B, D), x.dtype),
        in_specs=[
            pl.BlockSpec(memory_space=pl.ANY),
            pl.BlockSpec(memory_space=pl.ANY),
        ],
        out_specs=pl.BlockSpec(memory_space=pl.ANY),
        scratch_shapes=[pltpu.SemaphoreType.DMA],
        input_output_aliases={1: 0},
    )(t_emb, big)


# speedup vs baseline: 1.6063x; 1.4607x over previous
import jax
import jax.numpy as jnp
from jax.experimental import pallas as pl
from jax.experimental.pallas import tpu as pltpu

S, B, D = 2048, 4, 2048
BS = 256
N = S // BS


def _concat_embed_body(t_ref, x_ref, table_ref, out_ref, gat_sems):
    i = pl.program_id(0)

    @pl.when(i < N)
    def _copy():
        out_ref[...] = x_ref[...]

    @pl.when(i == N)
    def _embed():
        gathers = []
        for b in range(B):
            g = pltpu.make_async_copy(
                table_ref.at[t_ref[b]],
                out_ref.at[0, b],
                gat_sems.at[b],
            )
            g.start()
            gathers.append(g)
        for g in gathers:
            g.wait()


def kernel(x, t, table):
    return pl.pallas_call(
        _concat_embed_body,
        grid=(N + 1,),
        out_shape=jax.ShapeDtypeStruct((S + 1, B, D), x.dtype),
        in_specs=[
            pl.BlockSpec(memory_space=pltpu.SMEM),
            pl.BlockSpec((BS, B, D), lambda i: (jnp.minimum(i, N - 1), 0, 0)),
            pl.BlockSpec(memory_space=pl.ANY),
        ],
        out_specs=pl.BlockSpec((BS, B, D), lambda i: (i, 0, 0)),
        scratch_shapes=[
            pltpu.SemaphoreType.DMA((B,)),
        ],
    )(t, x, table)
